# baseline (speedup 1.0000x reference)
import jax
import jax.numpy as jnp
from jax.experimental import pallas as pl

_N = 10000
_H = 64
_BM = 400


def _big_body(h_ref, w1_ref, b_ref, adj_ref, z_ref):
    pooled = jax.lax.dot_general(
        adj_ref[...], h_ref[...], (((1,), (0,)), ((), ())),
        preferred_element_type=jnp.float32)
    z_ref[...] = jax.lax.dot_general(
        pooled, w1_ref[...], (((1,), (1,)), ((), ())),
        preferred_element_type=jnp.float32) + b_ref[...]


def _big_body0(h_ref, w1_ref, b_ref, adj_ref, z_ref):
    c = h_ref.shape[1]
    accs = [jnp.zeros((_BM, c), jnp.float32) for _ in range(8)]
    for i, k in enumerate(range(0, _N, 128)):
        e = min(k + 128, _N)
        part = jax.lax.dot_general(
            adj_ref[:, k:e], h_ref[k:e, :], (((1,), (0,)), ((), ())),
            preferred_element_type=jnp.float32)
        accs[i % 8] = accs[i % 8] + part
    pooled = accs[0]
    for a in accs[1:]:
        pooled = pooled + a
    z_ref[...] = jax.lax.dot_general(
        pooled, w1_ref[...], (((1,), (1,)), ((), ())),
        preferred_element_type=jnp.float32) + b_ref[...]


def _bn(v, g, b):
    mean = jnp.mean(v, axis=0, keepdims=True)
    var = jnp.mean((v - mean) ** 2, axis=0, keepdims=True)
    return (v - mean) / jnp.sqrt(var + 1e-5) * g + b


def _post(z_ref, w2_ref, b2_ref, g_ref, be_ref, og_ref, ob_ref):
    a = jnp.maximum(_bn(z_ref[...], g_ref[...], be_ref[...]), 0.0)
    u = jax.lax.dot_general(
        a, w2_ref[...], (((1,), (1,)), ((), ())),
        preferred_element_type=jnp.float32) + b2_ref[...]
    return jnp.maximum(_bn(u, og_ref[...], ob_ref[...]), 0.0)


def _mid_body(z_ref, w2_ref, b2_ref, g_ref, be_ref, og_ref, ob_ref, h_ref):
    hh = _post(z_ref, w2_ref, b2_ref, g_ref, be_ref, og_ref, ob_ref)
    h_ref[...] = hh.astype(jnp.bfloat16).astype(jnp.float32)


def _fin_body(z_ref, w2_ref, b2_ref, g_ref, be_ref, og_ref, ob_ref,
              gp_ref, h_ref, p_ref):
    hh = _post(z_ref, w2_ref, b2_ref, g_ref, be_ref, og_ref, ob_ref)
    h_ref[...] = hh
    p_ref[...] = jax.lax.dot_general(
        gp_ref[...], hh, (((1,), (0,)), ((), ())),
        preferred_element_type=jnp.float32)


def _big(h, w1, b, adj, layer0=False):
    c = h.shape[1]
    return pl.pallas_call(
        _big_body0 if layer0 else _big_body,
        grid=(_N // _BM,),
        in_specs=[
            pl.BlockSpec((_N, c), lambda i: (0, 0)),
            pl.BlockSpec((_H, c), lambda i: (0, 0)),
            pl.BlockSpec((1, _H), lambda i: (0, 0)),
            pl.BlockSpec((_BM, _N), lambda i: (i, 0)),
        ],
        out_specs=pl.BlockSpec((_BM, _H), lambda i: (i, 0)),
        out_shape=jax.ShapeDtypeStruct((_N, _H), jnp.float32),
    )(h, w1, b, adj)


def kernel(x, graph_pool, padded_nei, adj,
           l0_w1, l0_b1, l0_g1, l0_be1, l0_w2, l0_b2, l0_og, l0_ob,
           l1_w1, l1_b1, l1_g1, l1_be1, l1_w2, l1_b2, l1_og, l1_ob):
    del padded_nei
    r = lambda v: v.reshape(1, -1)

    z0 = _big(x, l0_w1, r(l0_b1), adj)

    h1 = pl.pallas_call(
        _mid_body,
        out_shape=jax.ShapeDtypeStruct((_N, _H), jnp.float32),
    )(z0, l0_w2, r(l0_b2), r(l0_g1), r(l0_be1), r(l0_og), r(l0_ob))

    z1 = _big(h1, l1_w1, r(l1_b1), adj)

    h2, pooled = pl.pallas_call(
        _fin_body,
        out_shape=[
            jax.ShapeDtypeStruct((_N, _H), jnp.float32),
            jax.ShapeDtypeStruct((graph_pool.shape[0], _H), jnp.float32),
        ],
    )(z1, l1_w2, r(l1_b2), r(l1_g1), r(l1_be1), r(l1_og), r(l1_ob),
      graph_pool)

    return (pooled, h2)
